# Initial kernel scaffold; baseline (speedup 1.0000x reference)
import jax
import jax.numpy as jnp
from jax.experimental import pallas as pl


def kernel(z, pos, batch, emb, mlp_w1, mlp_b1, mlp_w2, mlp_b2, lin1_w, lin2_w, lin2_b, lin_w, lin_b, out_w, out_b):
    raise NotImplementedError("write your pallas kernel here")


# speedup vs baseline: 11.4970x; 11.4970x over previous
import functools

import numpy as np
import jax
import jax.numpy as jnp
from jax.experimental import pallas as pl
from jax.experimental.pallas import tpu as pltpu

CUTOFF = 10.0
CHUNK = 128
N_GRAPHS = 128


def _ssp(x):
    return jax.nn.softplus(x) - jnp.log(2.0)


def _layer_body(meta_ref, pos_ref, batch_ref, h_ref,
                w1_ref, b1_ref, w2_ref, b2_ref,
                lin1_ref, lin2_ref, lin2b_ref, linw_ref, linb_ref,
                out_ref, *, n_gauss, coeff, gstep):
    c = pl.program_id(0)
    hid = h_ref.shape[1]
    p_r = pos_ref[pl.ds(c * CHUNK, CHUNK), :]
    h_r = h_ref[pl.ds(c * CHUNK, CHUNK), :]
    b_r = batch_ref[c, :]
    sq_r = jnp.sum(p_r * p_r, axis=1)
    block_lo = meta_ref[0, c]
    n_blocks = meta_ref[1, c]
    off3 = gstep * jax.lax.broadcasted_iota(
        jnp.int32, (1, 1, n_gauss), 2).astype(jnp.float32)

    row_ids = c * CHUNK + jax.lax.broadcasted_iota(jnp.int32, (CHUNK, CHUNK), 0)

    def body(t, acc):
        cb = block_lo + t
        p_c = pos_ref[pl.ds(cb * CHUNK, CHUNK), :]
        b_c = batch_ref[cb, :]
        h_c = h_ref[pl.ds(cb * CHUNK, CHUNK), :]
        x_c = jnp.dot(h_c, lin1_ref[...], preferred_element_type=jnp.float32)
        sq_c = jnp.sum(p_c * p_c, axis=1)
        d2 = (sq_r[:, None] + sq_c[None, :]
              - 2.0 * jnp.dot(p_r, p_c.T, preferred_element_type=jnp.float32))
        col_ids = cb * CHUNK + jax.lax.broadcasted_iota(
            jnp.int32, (CHUNK, CHUNK), 1)
        mask = ((b_r[:, None] == b_c[None, :])
                & (d2 <= CUTOFF * CUTOFF)
                & (row_ids != col_ids))
        ew = jnp.sqrt(jnp.maximum(d2, 0.0) + 1e-12)
        ccut = 0.5 * (jnp.cos(ew * (jnp.pi / CUTOFF)) + 1.0)
        cm = jnp.where(mask, ccut, 0.0)
        ea = jnp.exp(coeff * (ew[:, :, None] - off3) ** 2)
        ea2 = ea.reshape(CHUNK * CHUNK, n_gauss)
        t1 = _ssp(jnp.dot(ea2, w1_ref[...],
                          preferred_element_type=jnp.float32) + b1_ref[...])
        w = (jnp.dot(t1, w2_ref[...], preferred_element_type=jnp.float32)
             + b2_ref[...]).reshape(CHUNK, CHUNK, hid)
        w = w * cm[:, :, None]
        return acc + jnp.sum(w * x_c[None, :, :], axis=1)

    agg = jax.lax.fori_loop(0, n_blocks, body,
                            jnp.zeros((CHUNK, hid), jnp.float32))
    y = _ssp(jnp.dot(agg, lin2_ref[...], preferred_element_type=jnp.float32)
             + lin2b_ref[...])
    y = jnp.dot(y, linw_ref[...], preferred_element_type=jnp.float32) + linb_ref[...]
    out_ref[...] = h_r + y


def _readout_body(batchrow_ref, h_ref, out_w_ref, out_b_ref, out_ref):
    seg = jax.lax.broadcasted_iota(jnp.int32, (N_GRAPHS, h_ref.shape[0]), 0)
    m = (batchrow_ref[...] == seg).astype(jnp.float32)
    pooled = jnp.dot(m, h_ref[...], preferred_element_type=jnp.float32)
    out_ref[...] = (jnp.dot(pooled, out_w_ref[...],
                            preferred_element_type=jnp.float32)
                    + out_b_ref[...])


def kernel(z, pos, batch, emb, mlp_w1, mlp_b1, mlp_w2, mlp_b2,
           lin1_w, lin2_w, lin2_b, lin_w, lin_b, out_w, out_b):
    n = pos.shape[0]
    n_layers, n_gauss, filt = mlp_w1.shape
    hid = emb.shape[1]
    n_chunks = n // CHUNK
    offsets = np.linspace(0.0, CUTOFF, n_gauss)
    coeff = -0.5 / (float(CUTOFF) / (n_gauss - 1)) ** 2

    batch = batch.astype(jnp.int32)
    gids = jnp.arange(N_GRAPHS, dtype=jnp.int32)
    starts = jnp.searchsorted(batch, gids, side="left").astype(jnp.int32)
    ends = jnp.searchsorted(batch, gids, side="right").astype(jnp.int32)
    first_g = batch[:: CHUNK]
    last_g = batch[CHUNK - 1:: CHUNK]
    col_lo = starts[first_g]
    col_hi = ends[last_g]
    block_lo = col_lo // CHUNK
    block_hi = (col_hi - 1) // CHUNK
    meta = jnp.stack([block_lo, block_hi - block_lo + 1]).astype(jnp.int32)

    pos_pad = jnp.concatenate(
        [pos.astype(jnp.float32), jnp.zeros((n, 5), jnp.float32)], axis=1)
    batch2d = batch.reshape(n_chunks, CHUNK)
    h = emb[z]

    layer_call = pl.pallas_call(
        functools.partial(_layer_body, n_gauss=n_gauss,
                          coeff=coeff, gstep=float(CUTOFF) / (n_gauss - 1)),
        grid=(n_chunks,),
        in_specs=[
            pl.BlockSpec(memory_space=pltpu.SMEM),
            pl.BlockSpec((n, 8), lambda c: (0, 0)),
            pl.BlockSpec((n_chunks, CHUNK), lambda c: (0, 0)),
            pl.BlockSpec((n, hid), lambda c: (0, 0)),
            pl.BlockSpec((n_gauss, filt), lambda c: (0, 0)),
            pl.BlockSpec((1, filt), lambda c: (0, 0)),
            pl.BlockSpec((filt, filt), lambda c: (0, 0)),
            pl.BlockSpec((1, filt), lambda c: (0, 0)),
            pl.BlockSpec((hid, filt), lambda c: (0, 0)),
            pl.BlockSpec((filt, hid), lambda c: (0, 0)),
            pl.BlockSpec((1, hid), lambda c: (0, 0)),
            pl.BlockSpec((hid, hid), lambda c: (0, 0)),
            pl.BlockSpec((1, hid), lambda c: (0, 0)),
        ],
        out_specs=pl.BlockSpec((CHUNK, hid), lambda c: (c, 0)),
        out_shape=jax.ShapeDtypeStruct((n, hid), jnp.float32),
    )

    for l in range(n_layers):
        h = layer_call(meta, pos_pad, batch2d, h,
                       mlp_w1[l], mlp_b1[l][None, :], mlp_w2[l],
                       mlp_b2[l][None, :], lin1_w[l], lin2_w[l],
                       lin2_b[l][None, :], lin_w[l], lin_b[l][None, :])

    out = pl.pallas_call(
        _readout_body,
        in_specs=[
            pl.BlockSpec((1, n), lambda: (0, 0)),
            pl.BlockSpec((n, hid), lambda: (0, 0)),
            pl.BlockSpec((hid, out_w.shape[1]), lambda: (0, 0)),
            pl.BlockSpec((1, out_w.shape[1]), lambda: (0, 0)),
        ],
        out_specs=pl.BlockSpec((N_GRAPHS, out_w.shape[1]), lambda: (0, 0)),
        out_shape=jax.ShapeDtypeStruct((N_GRAPHS, out_w.shape[1]), jnp.float32),
    )(batch[None, :], h, out_w, out_b[None, :])
    return out


# speedup vs baseline: 30.5290x; 2.6554x over previous
import functools

import numpy as np
import jax
import jax.numpy as jnp
from jax.experimental import pallas as pl
from jax.experimental.pallas import tpu as pltpu

CUTOFF = 10.0
N_GRAPHS = 128
DST = 32
SRC = 64

_LOG2E = float(np.log2(np.e))
_LN2 = float(np.log(2.0))


def _ssp(x):
    return _LN2 * (jnp.log2(1.0 + jnp.exp2(x * _LOG2E)) - 1.0)


def _x_body(h_ref, lin1_ref, x_ref):
    x_ref[...] = jnp.dot(h_ref[...].astype(jnp.bfloat16), lin1_ref[...],
                         preferred_element_type=jnp.float32)


def _layer_body(meta_ref, pos_ref, bdst_ref, bsrc_ref, h_ref, x_ref,
                w1_ref, b1_ref, w2_ref, b2_ref,
                lin2_ref, lin2b_ref, linw_ref, linb_ref,
                out_ref, *, n_gauss, coeff, gstep):
    r = pl.program_id(0)
    hid = h_ref.shape[1]
    p_r = pos_ref[pl.ds(r * DST, DST), :]
    h_r = h_ref[pl.ds(r * DST, DST), :]
    b_r = bdst_ref[r, :]
    sq_r = jnp.sum(p_r * p_r, axis=1)
    block_lo = meta_ref[0, r]
    n_blocks = meta_ref[1, r]
    off3 = gstep * jax.lax.broadcasted_iota(
        jnp.int32, (1, 1, n_gauss), 2).astype(jnp.float32)

    row_ids = r * DST + jax.lax.broadcasted_iota(jnp.int32, (DST, 2 * SRC), 0)
    lane_in_hi = jax.lax.broadcasted_iota(jnp.int32, (DST, 2 * SRC), 1) >= SRC

    def body(t, acc):
        cb0 = block_lo + 2 * t
        hi_valid = 2 * t + 1 < n_blocks
        cb1 = jnp.where(hi_valid, cb0 + 1, block_lo)
        p_c = jnp.concatenate(
            [pos_ref[pl.ds(cb0 * SRC, SRC), :],
             pos_ref[pl.ds(cb1 * SRC, SRC), :]], axis=0)
        b_c = jnp.concatenate(
            [bsrc_ref[cb0, :], bsrc_ref[cb1, :]], axis=0)
        x_c = jnp.concatenate(
            [x_ref[pl.ds(cb0 * SRC, SRC), :],
             x_ref[pl.ds(cb1 * SRC, SRC), :]], axis=0)
        sq_c = jnp.sum(p_c * p_c, axis=1)
        d2 = (sq_r[:, None] + sq_c[None, :]
              - 2.0 * jnp.dot(p_r, p_c.T, preferred_element_type=jnp.float32))
        col_base = jnp.where(lane_in_hi, cb1 * SRC - SRC, cb0 * SRC)
        col_ids = col_base + jax.lax.broadcasted_iota(
            jnp.int32, (DST, 2 * SRC), 1)
        mask = ((b_r[:, None] == b_c[None, :])
                & (d2 <= CUTOFF * CUTOFF)
                & (row_ids != col_ids)
                & (hi_valid | ~lane_in_hi))
        ew = jnp.sqrt(jnp.maximum(d2, 0.0) + 1e-12)
        ccut = 0.5 * (jnp.cos(ew * (jnp.pi / CUTOFF)) + 1.0)
        cm = jnp.where(mask, ccut, 0.0)
        ea = jnp.exp(coeff * (ew[:, :, None] - off3) ** 2)
        ea2 = ea.reshape(DST * 2 * SRC, n_gauss).astype(jnp.bfloat16)
        t1 = _ssp(jnp.dot(ea2, w1_ref[...],
                          preferred_element_type=jnp.float32) + b1_ref[...])
        w = (jnp.dot(t1.astype(jnp.bfloat16), w2_ref[...],
                     preferred_element_type=jnp.float32)
             + b2_ref[...]).reshape(DST, 2 * SRC, hid)
        w = w * cm[:, :, None]
        return acc + jnp.sum(w * x_c[None, :, :], axis=1)

    agg = jax.lax.fori_loop(0, (n_blocks + 1) // 2, body,
                            jnp.zeros((DST, hid), jnp.float32))
    y = _ssp(jnp.dot(agg, lin2_ref[...], preferred_element_type=jnp.float32)
             + lin2b_ref[...])
    y = jnp.dot(y, linw_ref[...], preferred_element_type=jnp.float32) + linb_ref[...]
    out_ref[...] = h_r + y


def _readout_body(batchrow_ref, h_ref, out_w_ref, out_b_ref, out_ref):
    seg = jax.lax.broadcasted_iota(jnp.int32, (N_GRAPHS, h_ref.shape[0]), 0)
    m = (batchrow_ref[...] == seg).astype(jnp.float32)
    pooled = jnp.dot(m, h_ref[...], preferred_element_type=jnp.float32)
    out_ref[...] = (jnp.dot(pooled, out_w_ref[...],
                            preferred_element_type=jnp.float32)
                    + out_b_ref[...])


def kernel(z, pos, batch, emb, mlp_w1, mlp_b1, mlp_w2, mlp_b2,
           lin1_w, lin2_w, lin2_b, lin_w, lin_b, out_w, out_b):
    n = pos.shape[0]
    n_layers, n_gauss, filt = mlp_w1.shape
    hid = emb.shape[1]
    n_tiles = n // DST
    coeff = -0.5 / (float(CUTOFF) / (n_gauss - 1)) ** 2

    batch = batch.astype(jnp.int32)
    gids = jnp.arange(N_GRAPHS, dtype=jnp.int32)
    starts = jnp.searchsorted(batch, gids, side="left").astype(jnp.int32)
    ends = jnp.searchsorted(batch, gids, side="right").astype(jnp.int32)
    first_g = batch[::DST]
    last_g = batch[DST - 1::DST]
    col_lo = starts[first_g]
    col_hi = ends[last_g]
    block_lo = col_lo // SRC
    block_hi = (col_hi - 1) // SRC
    meta = jnp.stack([block_lo, block_hi - block_lo + 1]).astype(jnp.int32)

    pos_pad = jnp.concatenate(
        [pos.astype(jnp.float32), jnp.zeros((n, 5), jnp.float32)], axis=1)
    bdst = batch.reshape(n_tiles, DST)
    bsrc = batch.reshape(n // SRC, SRC)
    h = emb[z]

    x_call = pl.pallas_call(
        _x_body,
        in_specs=[pl.BlockSpec((n, hid), lambda: (0, 0)),
                  pl.BlockSpec((hid, filt), lambda: (0, 0))],
        out_specs=pl.BlockSpec((n, filt), lambda: (0, 0)),
        out_shape=jax.ShapeDtypeStruct((n, filt), jnp.float32),
    )

    layer_call = pl.pallas_call(
        functools.partial(_layer_body, n_gauss=n_gauss,
                          coeff=coeff, gstep=float(CUTOFF) / (n_gauss - 1)),
        grid=(n_tiles,),
        in_specs=[
            pl.BlockSpec(memory_space=pltpu.SMEM),
            pl.BlockSpec((n, 8), lambda r: (0, 0)),
            pl.BlockSpec((n_tiles, DST), lambda r: (0, 0)),
            pl.BlockSpec((n // SRC, SRC), lambda r: (0, 0)),
            pl.BlockSpec((n, hid), lambda r: (0, 0)),
            pl.BlockSpec((n, filt), lambda r: (0, 0)),
            pl.BlockSpec((n_gauss, filt), lambda r: (0, 0)),
            pl.BlockSpec((1, filt), lambda r: (0, 0)),
            pl.BlockSpec((filt, filt), lambda r: (0, 0)),
            pl.BlockSpec((1, filt), lambda r: (0, 0)),
            pl.BlockSpec((filt, hid), lambda r: (0, 0)),
            pl.BlockSpec((1, hid), lambda r: (0, 0)),
            pl.BlockSpec((hid, hid), lambda r: (0, 0)),
            pl.BlockSpec((1, hid), lambda r: (0, 0)),
        ],
        out_specs=pl.BlockSpec((DST, hid), lambda r: (r, 0)),
        out_shape=jax.ShapeDtypeStruct((n, hid), jnp.float32),
    )

    for l in range(n_layers):
        x = x_call(h, lin1_w[l].astype(jnp.bfloat16))
        h = layer_call(meta, pos_pad, bdst, bsrc, h, x,
                       mlp_w1[l].astype(jnp.bfloat16), mlp_b1[l][None, :],
                       mlp_w2[l].astype(jnp.bfloat16),
                       mlp_b2[l][None, :],
                       lin2_w[l], lin2_b[l][None, :], lin_w[l],
                       lin_b[l][None, :])

    out = pl.pallas_call(
        _readout_body,
        in_specs=[
            pl.BlockSpec((1, n), lambda: (0, 0)),
            pl.BlockSpec((n, hid), lambda: (0, 0)),
            pl.BlockSpec((hid, out_w.shape[1]), lambda: (0, 0)),
            pl.BlockSpec((1, out_w.shape[1]), lambda: (0, 0)),
        ],
        out_specs=pl.BlockSpec((N_GRAPHS, out_w.shape[1]), lambda: (0, 0)),
        out_shape=jax.ShapeDtypeStruct((N_GRAPHS, out_w.shape[1]), jnp.float32),
    )(batch[None, :], h, out_w, out_b[None, :])
    return out


# speedup vs baseline: 39.4100x; 1.2909x over previous
import functools

import numpy as np
import jax
import jax.numpy as jnp
from jax.experimental import pallas as pl
from jax.experimental.pallas import tpu as pltpu

CUTOFF = 10.0
N_GRAPHS = 128
DST = 32
SRC = 64

_LOG2E = float(np.log2(np.e))
_LN2 = float(np.log(2.0))


def _ssp(x):
    return _LN2 * (jnp.log2(1.0 + jnp.exp2(x * _LOG2E)) - 1.0)


def _x_body(h_ref, lin1_ref, x_ref):
    x_ref[...] = jnp.dot(h_ref[...].astype(jnp.bfloat16), lin1_ref[...],
                         preferred_element_type=jnp.float32)


def _post_body(h_ref, agg_ref, lin2_ref, lin2b_ref, linw_ref, linb_ref,
               lin1n_ref, h_out_ref, x_out_ref):
    y = _ssp(jnp.dot(agg_ref[...], lin2_ref[...],
                     preferred_element_type=jnp.float32) + lin2b_ref[...])
    y = (jnp.dot(y, linw_ref[...], preferred_element_type=jnp.float32)
         + linb_ref[...])
    h_new = h_ref[...] + y
    h_out_ref[...] = h_new
    x_out_ref[...] = jnp.dot(h_new.astype(jnp.bfloat16), lin1n_ref[...],
                             preferred_element_type=jnp.float32)


def _layer_body(meta_ref, pos_ref, bdst_ref, bsrc_ref, x_ref,
                w1_ref, b1_ref, w2_ref, b2_ref,
                out_ref, *, n_gauss, coeff, gstep):
    r = pl.program_id(0)
    hid = x_ref.shape[1]
    p_r = pos_ref[pl.ds(r * DST, DST), :]
    b_r = bdst_ref[r, :]
    sq_r = jnp.sum(p_r * p_r, axis=1)
    block_lo = meta_ref[0, r]
    n_blocks = meta_ref[1, r]
    off3 = gstep * jax.lax.broadcasted_iota(
        jnp.int32, (1, 1, n_gauss), 2).astype(jnp.float32)

    row_ids = r * DST + jax.lax.broadcasted_iota(jnp.int32, (DST, 2 * SRC), 0)
    lane_in_hi = jax.lax.broadcasted_iota(jnp.int32, (DST, 2 * SRC), 1) >= SRC

    def body(t, acc):
        cb0 = block_lo + 2 * t
        hi_valid = 2 * t + 1 < n_blocks
        cb1 = jnp.where(hi_valid, cb0 + 1, block_lo)
        p_c = jnp.concatenate(
            [pos_ref[pl.ds(cb0 * SRC, SRC), :],
             pos_ref[pl.ds(cb1 * SRC, SRC), :]], axis=0)
        b_c = jnp.concatenate(
            [bsrc_ref[cb0, :], bsrc_ref[cb1, :]], axis=0)
        x_c = jnp.concatenate(
            [x_ref[pl.ds(cb0 * SRC, SRC), :],
             x_ref[pl.ds(cb1 * SRC, SRC), :]], axis=0)
        sq_c = jnp.sum(p_c * p_c, axis=1)
        d2 = (sq_r[:, None] + sq_c[None, :]
              - 2.0 * jnp.dot(p_r, p_c.T, preferred_element_type=jnp.float32))
        col_base = jnp.where(lane_in_hi, cb1 * SRC - SRC, cb0 * SRC)
        col_ids = col_base + jax.lax.broadcasted_iota(
            jnp.int32, (DST, 2 * SRC), 1)
        mask = ((b_r[:, None] == b_c[None, :])
                & (d2 <= CUTOFF * CUTOFF)
                & (row_ids != col_ids)
                & (hi_valid | ~lane_in_hi))
        ew = jnp.sqrt(jnp.maximum(d2, 0.0) + 1e-12)
        ccut = 0.5 * (jnp.cos(ew * (jnp.pi / CUTOFF)) + 1.0)
        cm = jnp.where(mask, ccut, 0.0)
        offm = gstep * jax.lax.broadcasted_iota(
            jnp.int32, (1, n_gauss, 1), 1).astype(jnp.float32)
        ea = jnp.exp(coeff * (ew[:, None, :] - offm) ** 2).astype(jnp.bfloat16)
        t1p = jax.lax.dot_general(ea, w1_ref[...],
                                  (((1,), (0,)), ((), ())),
                                  preferred_element_type=jnp.float32)
        t1 = _ssp(t1p.reshape(DST * 2 * SRC, w1_ref.shape[1]) + b1_ref[...])
        w = (jnp.dot(t1.astype(jnp.bfloat16), w2_ref[...],
                     preferred_element_type=jnp.float32)
             + b2_ref[...]).reshape(DST, 2 * SRC, hid)
        w = w * cm[:, :, None]
        return acc + jnp.sum(w * x_c[None, :, :], axis=1)

    agg = jax.lax.fori_loop(0, (n_blocks + 1) // 2, body,
                            jnp.zeros((DST, hid), jnp.float32))
    out_ref[...] = agg


def _readout_body(batchrow_ref, h_ref, out_w_ref, out_b_ref, out_ref):
    seg = jax.lax.broadcasted_iota(jnp.int32, (N_GRAPHS, h_ref.shape[0]), 0)
    m = (batchrow_ref[...] == seg).astype(jnp.float32)
    pooled = jnp.dot(m, h_ref[...], preferred_element_type=jnp.float32)
    out_ref[...] = (jnp.dot(pooled, out_w_ref[...],
                            preferred_element_type=jnp.float32)
                    + out_b_ref[...])


def kernel(z, pos, batch, emb, mlp_w1, mlp_b1, mlp_w2, mlp_b2,
           lin1_w, lin2_w, lin2_b, lin_w, lin_b, out_w, out_b):
    n = pos.shape[0]
    n_layers, n_gauss, filt = mlp_w1.shape
    hid = emb.shape[1]
    n_tiles = n // DST
    coeff = -0.5 / (float(CUTOFF) / (n_gauss - 1)) ** 2

    batch = batch.astype(jnp.int32)
    gids = jnp.arange(N_GRAPHS, dtype=jnp.int32)
    starts = jnp.searchsorted(batch, gids, side="left").astype(jnp.int32)
    ends = jnp.searchsorted(batch, gids, side="right").astype(jnp.int32)
    first_g = batch[::DST]
    last_g = batch[DST - 1::DST]
    col_lo = starts[first_g]
    col_hi = ends[last_g]
    block_lo = col_lo // SRC
    block_hi = (col_hi - 1) // SRC
    meta = jnp.stack([block_lo, block_hi - block_lo + 1]).astype(jnp.int32)

    pos_pad = jnp.concatenate(
        [pos.astype(jnp.float32), jnp.zeros((n, 5), jnp.float32)], axis=1)
    bdst = batch.reshape(n_tiles, DST)
    bsrc = batch.reshape(n // SRC, SRC)
    h = emb[z]

    x_call = pl.pallas_call(
        _x_body,
        in_specs=[pl.BlockSpec((n, hid), lambda: (0, 0)),
                  pl.BlockSpec((hid, filt), lambda: (0, 0))],
        out_specs=pl.BlockSpec((n, filt), lambda: (0, 0)),
        out_shape=jax.ShapeDtypeStruct((n, filt), jnp.float32),
    )

    full = pl.BlockSpec((n, hid), lambda: (0, 0))
    row = pl.BlockSpec((1, hid), lambda: (0, 0))
    sq_spec = pl.BlockSpec((hid, hid), lambda: (0, 0))
    post_call = pl.pallas_call(
        _post_body,
        in_specs=[full, full, sq_spec, row, sq_spec, row, sq_spec],
        out_specs=(full, full),
        out_shape=(jax.ShapeDtypeStruct((n, hid), jnp.float32),
                   jax.ShapeDtypeStruct((n, filt), jnp.float32)),
    )

    layer_call = pl.pallas_call(
        functools.partial(_layer_body, n_gauss=n_gauss,
                          coeff=coeff, gstep=float(CUTOFF) / (n_gauss - 1)),
        grid=(n_tiles,),
        in_specs=[
            pl.BlockSpec(memory_space=pltpu.SMEM),
            pl.BlockSpec((n, 8), lambda r: (0, 0)),
            pl.BlockSpec((n_tiles, DST), lambda r: (0, 0)),
            pl.BlockSpec((n // SRC, SRC), lambda r: (0, 0)),
            pl.BlockSpec((n, filt), lambda r: (0, 0)),
            pl.BlockSpec((n_gauss, filt), lambda r: (0, 0)),
            pl.BlockSpec((1, filt), lambda r: (0, 0)),
            pl.BlockSpec((filt, filt), lambda r: (0, 0)),
            pl.BlockSpec((1, filt), lambda r: (0, 0)),
        ],
        out_specs=pl.BlockSpec((DST, hid), lambda r: (r, 0)),
        out_shape=jax.ShapeDtypeStruct((n, hid), jnp.float32),
    )

    x = x_call(h, lin1_w[0].astype(jnp.bfloat16))
    for l in range(n_layers):
        agg = layer_call(meta, pos_pad, bdst, bsrc, x,
                         mlp_w1[l].astype(jnp.bfloat16), mlp_b1[l][None, :],
                         mlp_w2[l].astype(jnp.bfloat16), mlp_b2[l][None, :])
        lin1_next = lin1_w[(l + 1) % n_layers]
        h, x = post_call(h, agg, lin2_w[l], lin2_b[l][None, :], lin_w[l],
                         lin_b[l][None, :], lin1_next.astype(jnp.bfloat16))

    out = pl.pallas_call(
        _readout_body,
        in_specs=[
            pl.BlockSpec((1, n), lambda: (0, 0)),
            pl.BlockSpec((n, hid), lambda: (0, 0)),
            pl.BlockSpec((hid, out_w.shape[1]), lambda: (0, 0)),
            pl.BlockSpec((1, out_w.shape[1]), lambda: (0, 0)),
        ],
        out_specs=pl.BlockSpec((N_GRAPHS, out_w.shape[1]), lambda: (0, 0)),
        out_shape=jax.ShapeDtypeStruct((N_GRAPHS, out_w.shape[1]), jnp.float32),
    )(batch[None, :], h, out_w, out_b[None, :])
    return out


# speedup vs baseline: 40.7653x; 1.0344x over previous
import functools

import numpy as np
import jax
import jax.numpy as jnp
from jax.experimental import pallas as pl
from jax.experimental.pallas import tpu as pltpu

CUTOFF = 10.0
N_GRAPHS = 128
DST = 32
SRC = 64

_LOG2E = float(np.log2(np.e))
_LN2 = float(np.log(2.0))


def _ssp(x):
    return _LN2 * (jnp.log2(1.0 + jnp.exp2(x * _LOG2E)) - 1.0)


def _x_body(h_ref, lin1_ref, x_ref):
    x_ref[...] = jnp.dot(h_ref[...].astype(jnp.bfloat16), lin1_ref[...],
                         preferred_element_type=jnp.float32)


def _post_body(h_ref, agg_ref, lin2_ref, lin2b_ref, linw_ref, linb_ref,
               lin1n_ref, h_out_ref, x_out_ref):
    y = _ssp(jnp.dot(agg_ref[...], lin2_ref[...],
                     preferred_element_type=jnp.float32) + lin2b_ref[...])
    y = (jnp.dot(y, linw_ref[...], preferred_element_type=jnp.float32)
         + linb_ref[...])
    h_new = h_ref[...] + y
    h_out_ref[...] = h_new
    x_out_ref[...] = jnp.dot(h_new.astype(jnp.bfloat16), lin1n_ref[...],
                             preferred_element_type=jnp.float32)


def _layer_body(meta_ref, pos_ref, srow_ref, erow_ref, x_ref,
                w1_ref, w2_ref, b2_ref,
                out_ref, *, n_gauss, coeff, gstep):
    r = pl.program_id(0)
    hid = x_ref.shape[1]
    WIN = 2 * SRC
    p_r = pos_ref[pl.ds(r * DST, DST), :]
    sq_r = jnp.sum(p_r * p_r, axis=1)
    s_col = srow_ref[r, :][:, None]
    e_col = erow_ref[r, :][:, None]
    w0 = meta_ref[0, r]
    nwin = meta_ref[1, r]
    last = meta_ref[2, r]

    row_ids = r * DST + jax.lax.broadcasted_iota(jnp.int32, (DST, WIN), 0)
    lane = jax.lax.broadcasted_iota(jnp.int32, (DST, WIN), 1)

    def body(t, acc):
        floor = w0 + WIN * t
        wstart = jnp.minimum(floor, last)
        p_c = pos_ref[pl.ds(wstart, WIN), :]
        x_c = x_ref[pl.ds(wstart, WIN), :]
        sq_c = jnp.sum(p_c * p_c, axis=1)
        d2 = (sq_r[:, None] + sq_c[None, :]
              - 2.0 * jnp.dot(p_r, p_c.T, preferred_element_type=jnp.float32))
        col_ids = wstart + lane
        mask = ((col_ids >= jnp.maximum(s_col, floor))
                & (col_ids < e_col)
                & (d2 <= CUTOFF * CUTOFF)
                & (row_ids != col_ids))
        ew = jnp.sqrt(jnp.maximum(d2, 0.0) + 1e-12)
        ccut = 0.5 * (jnp.cos(ew * (jnp.pi / CUTOFF)) + 1.0)
        cm = jnp.where(mask, ccut, 0.0)
        offm = gstep * jax.lax.broadcasted_iota(
            jnp.int32, (1, n_gauss, 1), 1).astype(jnp.float32)
        ea = jnp.exp(coeff * (ew[:, None, :] - offm) ** 2).astype(jnp.bfloat16)
        ea = jnp.concatenate(
            [ea, jnp.ones((DST, 1, 2 * SRC), jnp.bfloat16)], axis=1)
        t1p = jax.lax.dot_general(ea, w1_ref[...],
                                  (((1,), (0,)), ((), ())),
                                  preferred_element_type=jnp.float32)
        t1 = _ssp(t1p.reshape(DST * 2 * SRC,
                              w1_ref.shape[1])).astype(jnp.bfloat16)
        w = jnp.dot(t1, w2_ref[...],
                    preferred_element_type=jnp.float32).reshape(
                        DST, 2 * SRC, hid)
        w = w * cm[:, :, None]
        bias_term = b2_ref[...] * jnp.dot(cm, x_c,
                                          preferred_element_type=jnp.float32)
        return acc + jnp.sum(w * x_c[None, :, :], axis=1) + bias_term

    agg = jax.lax.fori_loop(0, nwin, body,
                            jnp.zeros((DST, hid), jnp.float32))
    out_ref[...] = agg


def _readout_body(batchrow_ref, h_ref, out_w_ref, out_b_ref, out_ref):
    seg = jax.lax.broadcasted_iota(jnp.int32, (N_GRAPHS, h_ref.shape[0]), 0)
    m = (batchrow_ref[...] == seg).astype(jnp.float32)
    pooled = jnp.dot(m, h_ref[...], preferred_element_type=jnp.float32)
    out_ref[...] = (jnp.dot(pooled, out_w_ref[...],
                            preferred_element_type=jnp.float32)
                    + out_b_ref[...])


def kernel(z, pos, batch, emb, mlp_w1, mlp_b1, mlp_w2, mlp_b2,
           lin1_w, lin2_w, lin2_b, lin_w, lin_b, out_w, out_b):
    n = pos.shape[0]
    n_layers, n_gauss, filt = mlp_w1.shape
    hid = emb.shape[1]
    n_tiles = n // DST
    coeff = -0.5 / (float(CUTOFF) / (n_gauss - 1)) ** 2

    batch = batch.astype(jnp.int32)
    gids = jnp.arange(N_GRAPHS, dtype=jnp.int32)
    starts = jnp.searchsorted(batch, gids, side="left").astype(jnp.int32)
    ends = jnp.searchsorted(batch, gids, side="right").astype(jnp.int32)
    first_g = batch[::DST]
    last_g = batch[DST - 1::DST]
    col_lo = starts[first_g]
    col_hi = ends[last_g]
    WIN = 2 * SRC
    w0 = (col_lo // 8) * 8
    nwin = (col_hi - w0 + WIN - 1) // WIN
    last = jnp.clip(((col_hi + 7) // 8) * 8 - WIN, 0, n - WIN)
    meta = jnp.stack([w0, nwin, last]).astype(jnp.int32)
    srow = starts[batch].reshape(n_tiles, DST)
    erow = ends[batch].reshape(n_tiles, DST)

    pos_pad = jnp.concatenate(
        [pos.astype(jnp.float32), jnp.zeros((n, 5), jnp.float32)], axis=1)
    h = emb[z]

    x_call = pl.pallas_call(
        _x_body,
        in_specs=[pl.BlockSpec((n, hid), lambda: (0, 0)),
                  pl.BlockSpec((hid, filt), lambda: (0, 0))],
        out_specs=pl.BlockSpec((n, filt), lambda: (0, 0)),
        out_shape=jax.ShapeDtypeStruct((n, filt), jnp.float32),
    )

    full = pl.BlockSpec((n, hid), lambda: (0, 0))
    row = pl.BlockSpec((1, hid), lambda: (0, 0))
    sq_spec = pl.BlockSpec((hid, hid), lambda: (0, 0))
    post_call = pl.pallas_call(
        _post_body,
        in_specs=[full, full, sq_spec, row, sq_spec, row, sq_spec],
        out_specs=(full, full),
        out_shape=(jax.ShapeDtypeStruct((n, hid), jnp.float32),
                   jax.ShapeDtypeStruct((n, filt), jnp.float32)),
    )

    layer_call = pl.pallas_call(
        functools.partial(_layer_body, n_gauss=n_gauss,
                          coeff=coeff, gstep=float(CUTOFF) / (n_gauss - 1)),
        grid=(n_tiles,),
        in_specs=[
            pl.BlockSpec(memory_space=pltpu.SMEM),
            pl.BlockSpec((n, 8), lambda r: (0, 0)),
            pl.BlockSpec((n_tiles, DST), lambda r: (0, 0)),
            pl.BlockSpec((n_tiles, DST), lambda r: (0, 0)),
            pl.BlockSpec((n, filt), lambda r: (0, 0)),
            pl.BlockSpec((n_gauss + 1, filt), lambda r: (0, 0)),
            pl.BlockSpec((filt, filt), lambda r: (0, 0)),
            pl.BlockSpec((1, filt), lambda r: (0, 0)),
        ],
        out_specs=pl.BlockSpec((DST, hid), lambda r: (r, 0)),
        out_shape=jax.ShapeDtypeStruct((n, hid), jnp.float32),
    )

    x = x_call(h, lin1_w[0].astype(jnp.bfloat16))
    for l in range(n_layers):
        w1a = jnp.concatenate([mlp_w1[l], mlp_b1[l][None, :]], axis=0)
        agg = layer_call(meta, pos_pad, srow, erow, x,
                         w1a.astype(jnp.bfloat16),
                         mlp_w2[l].astype(jnp.bfloat16), mlp_b2[l][None, :])
        lin1_next = lin1_w[(l + 1) % n_layers]
        h, x = post_call(h, agg, lin2_w[l], lin2_b[l][None, :], lin_w[l],
                         lin_b[l][None, :], lin1_next.astype(jnp.bfloat16))

    out = pl.pallas_call(
        _readout_body,
        in_specs=[
            pl.BlockSpec((1, n), lambda: (0, 0)),
            pl.BlockSpec((n, hid), lambda: (0, 0)),
            pl.BlockSpec((hid, out_w.shape[1]), lambda: (0, 0)),
            pl.BlockSpec((1, out_w.shape[1]), lambda: (0, 0)),
        ],
        out_specs=pl.BlockSpec((N_GRAPHS, out_w.shape[1]), lambda: (0, 0)),
        out_shape=jax.ShapeDtypeStruct((N_GRAPHS, out_w.shape[1]), jnp.float32),
    )(batch[None, :], h, out_w, out_b[None, :])
    return out
